# baseline (speedup 1.0000x reference)
import functools

import jax
import jax.numpy as jnp
from jax.experimental import pallas as pl
from jax.experimental.pallas import tpu as pltpu

_BM = 400


def _gcn_kernel(x_ref, w1_ref, w2_ref, adjl_ref, adjr_ref, out_ref,
                s1_ref, s2_ref, *, nb, n2):
    i = pl.program_id(0)

    @pl.when(i == 0)
    def _prologue():
        s1_ref[...] = jnp.dot(x_ref[...], w1_ref[...],
                              preferred_element_type=jnp.float32)

    def _adj_dot(rhs_ref):
        return (jnp.dot(adjl_ref[:, 0, 0, :], rhs_ref[:n2, :],
                        preferred_element_type=jnp.float32)
                + jnp.dot(adjr_ref[:, 0, 0, :], rhs_ref[n2:, :],
                          preferred_element_type=jnp.float32))

    @pl.when(i < nb)
    def _layer1():
        h = jnp.maximum(_adj_dot(s1_ref), 0.0)
        s2 = jnp.dot(h, w2_ref[...], preferred_element_type=jnp.float32)
        s2_ref[pl.ds((i % nb) * _BM, _BM), :] = s2

    @pl.when(i >= nb)
    def _layer2():
        out_ref[...] = _adj_dot(s2_ref)


@jax.jit
def kernel(x, adj, W1, W2):
    n, nfeat = x.shape
    nhid = W1.shape[1]
    nout = W2.shape[1]
    nb = n // _BM

    n2 = n // 2
    adj4 = adj.reshape(n, 2, 1, n2)
    return pl.pallas_call(
        functools.partial(_gcn_kernel, nb=nb, n2=n2),
        grid=(2 * nb,),
        in_specs=[
            pl.BlockSpec((n, nfeat), lambda i: (0, 0)),
            pl.BlockSpec((nfeat, nhid), lambda i: (0, 0)),
            pl.BlockSpec((nhid, nout), lambda i: (0, 0)),
            pl.BlockSpec((_BM, 1, 1, n2), lambda i, nb=nb: (i % nb, 0, 0, 0)),
            pl.BlockSpec((_BM, 1, 1, n2), lambda i, nb=nb: (i % nb, 1, 0, 0)),
        ],
        out_specs=pl.BlockSpec(
            (_BM, nout),
            lambda i, nb=nb: (jnp.where(i >= nb, i - nb, 0), 0)),
        out_shape=jax.ShapeDtypeStruct((n, nout), jnp.float32),
        scratch_shapes=[
            pltpu.VMEM((n, nhid), jnp.float32),
            pltpu.VMEM((n, nout), jnp.float32),
        ],
        compiler_params=pltpu.CompilerParams(
            vmem_limit_bytes=100 * 1024 * 1024,
        ),
    )(x, W1, W2, adj4, adj4)


# speedup vs baseline: 12.7526x; 12.7526x over previous
import functools

import jax
import jax.numpy as jnp
from jax.experimental import pallas as pl
from jax.experimental.pallas import tpu as pltpu

_BM = 400


def _gcn_kernel(x_ref, w1_ref, w2_ref, adj_ref, out_ref, s1_ref, s2_ref):
    p = pl.program_id(0)
    b = pl.program_id(1)

    @pl.when(jnp.logical_and(p == 0, b == 0))
    def _prologue():
        s1_ref[...] = jnp.dot(x_ref[...], w1_ref[...],
                              preferred_element_type=jnp.float32)

    @pl.when(p == 0)
    def _layer1():
        h = jnp.dot(adj_ref[...], s1_ref[...],
                    preferred_element_type=jnp.float32)
        h = jnp.maximum(h, 0.0)
        s2 = jnp.dot(h, w2_ref[...], preferred_element_type=jnp.float32)
        s2_ref[pl.ds(b * _BM, _BM), :] = s2

    @pl.when(p == 1)
    def _layer2():
        out_ref[...] = jnp.dot(adj_ref[...], s2_ref[...],
                               preferred_element_type=jnp.float32)


@jax.jit
def kernel(x, adj, W1, W2):
    n, nfeat = x.shape
    nhid = W1.shape[1]
    nout = W2.shape[1]
    nb = n // _BM

    return pl.pallas_call(
        _gcn_kernel,
        grid=(2, nb),
        in_specs=[
            pl.BlockSpec((n, nfeat), lambda p, b: (0, 0)),
            pl.BlockSpec((nfeat, nhid), lambda p, b: (0, 0)),
            pl.BlockSpec((nhid, nout), lambda p, b: (0, 0)),
            pl.BlockSpec((_BM, n), lambda p, b: (b, 0)),
        ],
        out_specs=pl.BlockSpec((_BM, nout), lambda p, b: (b, 0)),
        out_shape=jax.ShapeDtypeStruct((n, nout), jnp.float32),
        scratch_shapes=[
            pltpu.VMEM((n, nhid), jnp.float32),
            pltpu.VMEM((n, nout), jnp.float32),
        ],
        compiler_params=pltpu.CompilerParams(
            vmem_limit_bytes=100 * 1024 * 1024,
        ),
    )(x, W1, W2, adj)
